# Initial kernel scaffold; baseline (speedup 1.0000x reference)
import jax
import jax.numpy as jnp
from jax.experimental import pallas as pl


def kernel(x, edge_index, W, b):
    raise NotImplementedError("write your pallas kernel here")


# speedup vs baseline: 6.5547x; 6.5547x over previous
import functools

import jax
import jax.numpy as jnp
from jax import lax
from jax.experimental import pallas as pl
from jax.experimental.pallas import tpu as pltpu
from jax.experimental.pallas import tpu_sc as plsc

D = 128
DW = 144
NC, NS = 2, 16
NW = NC * NS
CHUNK = 128


def _sc_aggregate(xw, src, dst, n_pad, c_per_tile):
    rows_per_tile = n_pad // NS
    mesh = plsc.VectorSubcoreMesh(core_axis_name="c", subcore_axis_name="s")

    @functools.partial(
        pl.kernel,
        out_type=jax.ShapeDtypeStruct((NC, n_pad, DW), jnp.float32),
        mesh=mesh,
        compiler_params=pltpu.CompilerParams(use_tc_tiling_on_sc=False),
        scratch_types=[
            pltpu.VMEM((CHUNK,), jnp.int32),
            pltpu.VMEM((CHUNK,), jnp.int32),
            pltpu.VMEM((CHUNK, DW), jnp.float32),
            pltpu.VMEM_SHARED((n_pad, DW), jnp.float32),
            pltpu.SemaphoreType.DMA,
        ],
    )
    def k(xw_hbm, src_hbm, dst_hbm, out_hbm, idx_s, idx_d, rows, acc, sem):
        cid = lax.axis_index("c")
        sid = lax.axis_index("s")
        wid = cid * NS + sid

        @pl.loop(0, CHUNK)
        def _(r):
            @pl.loop(0, DW // 16)
            def _(cc):
                rows.at[pl.ds(r, 1), pl.ds(cc * 16, 16)][...] = (
                    jnp.zeros((1, 16), jnp.float32))

        @pl.loop(0, rows_per_tile // CHUNK)
        def _(kk):
            pltpu.sync_copy(
                rows, acc.at[pl.ds(sid * rows_per_tile + kk * CHUNK, CHUNK)])

        plsc.subcore_barrier()

        base = wid * (c_per_tile * CHUNK)

        @pl.loop(0, c_per_tile)
        def _(ci):
            off = base + ci * CHUNK
            pltpu.sync_copy(src_hbm.at[pl.ds(off, CHUNK)], idx_s)
            pltpu.sync_copy(dst_hbm.at[pl.ds(off, CHUNK)], idx_d)
            pltpu.async_copy(xw_hbm.at[idx_s], rows, sem).wait()
            pltpu.sync_copy(rows, acc.at[idx_d], add=True)

        plsc.subcore_barrier()

        @pl.loop(0, rows_per_tile // CHUNK)
        def _(h):
            r0 = sid * rows_per_tile + h * CHUNK
            pltpu.sync_copy(acc.at[pl.ds(r0, CHUNK)], rows)
            pltpu.sync_copy(rows, out_hbm.at[cid, pl.ds(r0, CHUNK)])

    return k(xw, src, dst)


def _tc_update(partials, xw, wt, b2, n_pad):
    blk = 1024
    grid = n_pad // blk

    def body(p_ref, xw_ref, wt_ref, b_ref, o_ref):
        s = p_ref[0] + p_ref[1] + xw_ref[...]
        cnt = jnp.maximum(s[:, D:D + 1], 1.0)
        aggr = s[:, :D] / cnt
        out = jnp.dot(aggr, wt_ref[...],
                      preferred_element_type=jnp.float32) + b_ref[...]
        nrm = jnp.sqrt(jnp.sum(out * out, axis=1, keepdims=True))
        o_ref[...] = out / jnp.maximum(nrm, 1e-12)

    return pl.pallas_call(
        body,
        grid=(grid,),
        in_specs=[
            pl.BlockSpec((NC, blk, DW), lambda i: (0, i, 0)),
            pl.BlockSpec((blk, DW), lambda i: (i, 0)),
            pl.BlockSpec((D, D), lambda i: (0, 0)),
            pl.BlockSpec((1, D), lambda i: (0, 0)),
        ],
        out_specs=pl.BlockSpec((blk, D), lambda i: (i, 0)),
        out_shape=jax.ShapeDtypeStruct((n_pad, D), jnp.float32),
    )(partials, xw, wt, b2)


def kernel(x, edge_index, W, b):
    n = x.shape[0]
    e = edge_index.shape[1]
    n_pad = ((n + 1 + 1023) // 1024) * 1024
    c_per_tile = (e + CHUNK * NW - 1) // (CHUNK * NW)
    e_pad = c_per_tile * CHUNK * NW

    src = edge_index[0].astype(jnp.int32)
    dst = edge_index[1].astype(jnp.int32)
    if e_pad > e:
        src = jnp.concatenate([src, jnp.zeros((e_pad - e,), jnp.int32)])
        dst = jnp.concatenate([dst, jnp.full((e_pad - e,), n, jnp.int32)])

    xw = jnp.concatenate([x, jnp.ones((n, DW - D), jnp.float32)], axis=1)
    xw = jnp.pad(xw, ((0, n_pad - n), (0, 0)))

    partials = _sc_aggregate(xw, src, dst, n_pad, c_per_tile)
    out = _tc_update(partials, xw, W.T, b.reshape(1, D), n_pad)
    return out[:n]
